# Initial kernel scaffold; baseline (speedup 1.0000x reference)
import jax
import jax.numpy as jnp
from jax.experimental import pallas as pl


def kernel(x, edge_index, W1, b1, W2, b2):
    raise NotImplementedError("write your pallas kernel here")


# speedup vs baseline: 10.3651x; 10.3651x over previous
import functools

import jax
import jax.numpy as jnp
from jax import lax
from jax.experimental import pallas as pl
from jax.experimental.pallas import tpu as pltpu
from jax.experimental.pallas import tpu_sc as plsc

N = 10000
E = 160000
D_IN = 256
D_HID = 64
D_CLS = 40

NC = 2
NS = 16
NW = NC * NS
CHUNK = 128
CPT = -(-E // (NW * CHUNK))
E_PAD = NW * CHUNK * CPT
ACC_ROWS = 10240

_mesh = plsc.VectorSubcoreMesh(core_axis_name="c", subcore_axis_name="s")


@functools.partial(
    pl.kernel,
    out_type=jax.ShapeDtypeStruct((NC, ACC_ROWS, 16), jnp.float32),
    mesh=_mesh,
    scratch_types=[
        pltpu.VMEM((CHUNK,), jnp.int32),
        pltpu.VMEM((CHUNK, 16), jnp.float32),
        pltpu.VMEM_SHARED((ACC_ROWS, 16), jnp.float32),
        pltpu.SemaphoreType.DMA,
    ],
)
def _deg_kernel(dst_hbm, ones_hbm, zeros_hbm, out_hbm, idx_v, ones_v, acc, sem):
    cid = lax.axis_index("c")
    sid = lax.axis_index("s")
    wid = sid * NC + cid
    rows_per = ACC_ROWS // NS
    pltpu.sync_copy(zeros_hbm.at[pl.ds(sid * rows_per, rows_per)],
                    acc.at[pl.ds(sid * rows_per, rows_per)])
    pltpu.sync_copy(ones_hbm, ones_v)
    plsc.subcore_barrier()

    def body(i, carry):
        base = (wid * CPT + i) * CHUNK
        pltpu.sync_copy(dst_hbm.at[pl.ds(base, CHUNK)], idx_v)
        pltpu.sync_copy(ones_v, acc.at[idx_v], add=True)
        return carry

    lax.fori_loop(0, CPT, body, 0)
    plsc.subcore_barrier()
    pltpu.sync_copy(acc.at[pl.ds(sid * rows_per, rows_per)],
                    out_hbm.at[cid, pl.ds(sid * rows_per, rows_per)])


def _make_scatter(d):
    @functools.partial(
        pl.kernel,
        out_type=jax.ShapeDtypeStruct((NC, ACC_ROWS, d), jnp.float32),
        mesh=_mesh,
        scratch_types=[
            pltpu.VMEM((CHUNK,), jnp.int32),
            pltpu.VMEM((CHUNK,), jnp.int32),
            pltpu.VMEM((CHUNK, d), jnp.float32),
            pltpu.VMEM_SHARED((ACC_ROWS, d), jnp.float32),
            pltpu.SemaphoreType.DMA,
        ],
        compiler_params=pltpu.CompilerParams(use_tc_tiling_on_sc=False),
    )
    def _scatter_kernel(src_hbm, dst_hbm, g_hbm, zeros_hbm, out_hbm,
                        src_v, dst_v, rows_v, acc, sem):
        cid = lax.axis_index("c")
        sid = lax.axis_index("s")
        wid = sid * NC + cid
        rows_per = ACC_ROWS // NS
        pltpu.sync_copy(zeros_hbm.at[pl.ds(sid * rows_per, rows_per)],
                        acc.at[pl.ds(sid * rows_per, rows_per)])
        plsc.subcore_barrier()

        def body(i, carry):
            base = (wid * CPT + i) * CHUNK
            pltpu.sync_copy(src_hbm.at[pl.ds(base, CHUNK)], src_v)
            pltpu.sync_copy(dst_hbm.at[pl.ds(base, CHUNK)], dst_v)
            pltpu.async_copy(g_hbm.at[src_v], rows_v, sem).wait()
            pltpu.sync_copy(rows_v, acc.at[dst_v], add=True)
            return carry

        lax.fori_loop(0, CPT, body, 0)
        plsc.subcore_barrier()
        pltpu.sync_copy(acc.at[pl.ds(sid * rows_per, rows_per)],
                        out_hbm.at[cid, pl.ds(sid * rows_per, rows_per)])

    return _scatter_kernel


_scatter64 = _make_scatter(D_HID)
_scatter40 = _make_scatter(D_CLS)


BM = 2000


def _tc1_body(x_ref, w1_ref, dp_ref, g1_ref, dis_ref):
    deg = dp_ref[0][:, 0:1] + dp_ref[1][:, 0:1] + 1.0
    dis = lax.rsqrt(deg)
    h = jnp.dot(x_ref[...], w1_ref[...], preferred_element_type=jnp.float32)
    g1_ref[...] = h * dis
    dis_ref[...] = dis


def _tc2_body(p_ref, g1_ref, dis_ref, b1_ref, w2_ref, g2_ref):
    dis = dis_ref[...]
    s = (p_ref[0] + p_ref[1] + g1_ref[...]) * dis + b1_ref[...]
    h1 = jnp.maximum(s, 0.0)
    g2_ref[...] = jnp.dot(h1, w2_ref[...], preferred_element_type=jnp.float32) * dis


def _tc3_body(p_ref, g2_ref, dis_ref, b2_ref, out_ref):
    z = (p_ref[0] + p_ref[1] + g2_ref[...]) * dis_ref[...] + b2_ref[...]
    m = jnp.max(z, axis=1, keepdims=True)
    zs = z - m
    out_ref[...] = zs - jnp.log(jnp.sum(jnp.exp(zs), axis=1, keepdims=True))


def _tc1(x, w1, deg_parts):
    grid = N // BM
    return pl.pallas_call(
        _tc1_body,
        grid=(grid,),
        in_specs=[
            pl.BlockSpec((BM, D_IN), lambda i: (i, 0)),
            pl.BlockSpec((D_IN, D_HID), lambda i: (0, 0)),
            pl.BlockSpec((NC, BM, 16), lambda i: (0, i, 0)),
        ],
        out_specs=[
            pl.BlockSpec((BM, D_HID), lambda i: (i, 0)),
            pl.BlockSpec((BM, 1), lambda i: (i, 0)),
        ],
        out_shape=[
            jax.ShapeDtypeStruct((N, D_HID), jnp.float32),
            jax.ShapeDtypeStruct((N, 1), jnp.float32),
        ],
    )(x, w1, deg_parts)


def _tc2(parts1, g1, dis, b1, w2):
    grid = N // BM
    return pl.pallas_call(
        _tc2_body,
        grid=(grid,),
        in_specs=[
            pl.BlockSpec((NC, BM, D_HID), lambda i: (0, i, 0)),
            pl.BlockSpec((BM, D_HID), lambda i: (i, 0)),
            pl.BlockSpec((BM, 1), lambda i: (i, 0)),
            pl.BlockSpec((1, D_HID), lambda i: (0, 0)),
            pl.BlockSpec((D_HID, D_CLS), lambda i: (0, 0)),
        ],
        out_specs=pl.BlockSpec((BM, D_CLS), lambda i: (i, 0)),
        out_shape=jax.ShapeDtypeStruct((N, D_CLS), jnp.float32),
    )(parts1, g1, dis, b1, w2)


def _tc3(parts2, g2, dis, b2):
    grid = N // BM
    return pl.pallas_call(
        _tc3_body,
        grid=(grid,),
        in_specs=[
            pl.BlockSpec((NC, BM, D_CLS), lambda i: (0, i, 0)),
            pl.BlockSpec((BM, D_CLS), lambda i: (i, 0)),
            pl.BlockSpec((BM, 1), lambda i: (i, 0)),
            pl.BlockSpec((1, D_CLS), lambda i: (0, 0)),
        ],
        out_specs=pl.BlockSpec((BM, D_CLS), lambda i: (i, 0)),
        out_shape=jax.ShapeDtypeStruct((N, D_CLS), jnp.float32),
    )(parts2, g2, dis, b2)


@jax.jit
def kernel(x, edge_index, W1, b1, W2, b2):
    pad = E_PAD - E
    src_p = jnp.concatenate([edge_index[0], jnp.zeros((pad,), jnp.int32)])
    dst_p = jnp.concatenate([edge_index[1], jnp.full((pad,), N, jnp.int32)])

    ones16 = jnp.ones((CHUNK, 16), jnp.float32)
    zeros16 = jnp.zeros((ACC_ROWS, 16), jnp.float32)
    zeros64 = jnp.zeros((ACC_ROWS, D_HID), jnp.float32)
    zeros40 = jnp.zeros((ACC_ROWS, D_CLS), jnp.float32)

    deg_parts = _deg_kernel(dst_p, ones16, zeros16)
    g1, dis = _tc1(x, W1, deg_parts)
    parts1 = _scatter64(src_p, dst_p, g1, zeros64)
    g2 = _tc2(parts1, g1, dis, b1.reshape(1, D_HID), W2)
    parts2 = _scatter40(src_p, dst_p, g2, zeros40)
    return _tc3(parts2, g2, dis, b2.reshape(1, D_CLS))


# speedup vs baseline: 24.7856x; 2.3913x over previous
import functools

import jax
import jax.numpy as jnp
from jax import lax
from jax.experimental import pallas as pl
from jax.experimental.pallas import tpu as pltpu
from jax.experimental.pallas import tpu_sc as plsc

N = 10000
E = 160000
D_IN = 256
D_HID = 64
D_CLS = 40

NC = 2
NS = 16
NW = NC * NS
CHUNK = 128
CPT = -(-E // (NW * CHUNK))
E_PAD = NW * CHUNK * CPT
ACC_ROWS = 10240

_mesh = plsc.VectorSubcoreMesh(core_axis_name="c", subcore_axis_name="s")
_sc_params = pltpu.CompilerParams(use_tc_tiling_on_sc=False)


@functools.partial(
    pl.kernel,
    out_type=jax.ShapeDtypeStruct((NC, ACC_ROWS, 16), jnp.float32),
    mesh=_mesh,
    scratch_types=[
        pltpu.VMEM((CPT, CHUNK), jnp.int32),
        pltpu.VMEM((CHUNK, 16), jnp.float32),
        pltpu.VMEM_SHARED((ACC_ROWS, 16), jnp.float32),
        pltpu.SemaphoreType.DMA,
    ],
    compiler_params=_sc_params,
)
def _deg_kernel(dst_hbm, ones_hbm, zeros_hbm, out_hbm, dst_v, ones_v, acc, sem):
    cid = lax.axis_index("c")
    sid = lax.axis_index("s")
    wid = sid * NC + cid
    rows_per = ACC_ROWS // NS
    pltpu.sync_copy(zeros_hbm.at[pl.ds(sid * rows_per, rows_per)],
                    acc.at[pl.ds(sid * rows_per, rows_per)])
    pltpu.sync_copy(dst_hbm.at[wid], dst_v)
    pltpu.sync_copy(ones_hbm, ones_v)
    plsc.subcore_barrier()

    for j in range(4):
        pltpu.async_copy(ones_v, acc.at[dst_v.at[j]], sem, add=True)

    def body(j, carry):
        pltpu.make_async_copy(ones_v, acc.at[dst_v.at[j]], sem).wait()

        @pl.when(j + 4 < CPT)
        def _():
            pltpu.async_copy(ones_v, acc.at[dst_v.at[j + 4]], sem, add=True)

        return carry

    lax.fori_loop(0, CPT, body, 0)
    plsc.subcore_barrier()
    pltpu.sync_copy(acc.at[pl.ds(sid * rows_per, rows_per)],
                    out_hbm.at[cid, pl.ds(sid * rows_per, rows_per)])


def _make_scatter(d):
    @functools.partial(
        pl.kernel,
        out_type=jax.ShapeDtypeStruct((NC, ACC_ROWS, d), jnp.float32),
        mesh=_mesh,
        scratch_types=[
            pltpu.VMEM((CPT, CHUNK), jnp.int32),
            pltpu.VMEM((CPT, CHUNK), jnp.int32),
            pltpu.VMEM((2, CHUNK, d), jnp.float32),
            pltpu.VMEM_SHARED((ACC_ROWS, d), jnp.float32),
            pltpu.SemaphoreType.DMA,
        ],
        compiler_params=_sc_params,
    )
    def _scatter_kernel(src_hbm, dst_hbm, g_hbm, zeros_hbm, out_hbm,
                        src_v, dst_v, rows_v, acc, gsem):
        cid = lax.axis_index("c")
        sid = lax.axis_index("s")
        wid = sid * NC + cid
        rows_per = ACC_ROWS // NS
        pltpu.sync_copy(zeros_hbm.at[pl.ds(sid * rows_per, rows_per)],
                        acc.at[pl.ds(sid * rows_per, rows_per)])
        pltpu.sync_copy(src_hbm.at[wid], src_v)
        pltpu.sync_copy(dst_hbm.at[wid], dst_v)
        plsc.subcore_barrier()

        pltpu.async_copy(g_hbm.at[src_v.at[0]], rows_v.at[0], gsem)

        def body(j, carry):
            @pl.when(j + 1 < CPT)
            def _():
                pltpu.async_copy(g_hbm.at[src_v.at[j + 1]],
                                 rows_v.at[(j + 1) % 2], gsem)

            pltpu.make_async_copy(g_hbm.at[src_v.at[j]],
                                  rows_v.at[j % 2], gsem).wait()
            pltpu.sync_copy(rows_v.at[j % 2], acc.at[dst_v.at[j]], add=True)
            return carry

        lax.fori_loop(0, CPT, body, 0)
        plsc.subcore_barrier()
        pltpu.sync_copy(acc.at[pl.ds(sid * rows_per, rows_per)],
                        out_hbm.at[cid, pl.ds(sid * rows_per, rows_per)])

    return _scatter_kernel


_scatter64 = _make_scatter(D_HID)
_scatter40 = _make_scatter(D_CLS)


BM = 2000


def _tc1_body(x_ref, w1_ref, dp_ref, g1_ref, dis_ref):
    deg = dp_ref[0][:, 0:1] + dp_ref[1][:, 0:1] + 1.0
    dis = lax.rsqrt(deg)
    h = jnp.dot(x_ref[...], w1_ref[...], preferred_element_type=jnp.float32)
    g1_ref[...] = h * dis
    dis_ref[...] = dis


def _tc2_body(p_ref, g1_ref, dis_ref, b1_ref, w2_ref, g2_ref):
    dis = dis_ref[...]
    s = (p_ref[0] + p_ref[1] + g1_ref[...]) * dis + b1_ref[...]
    h1 = jnp.maximum(s, 0.0)
    g2_ref[...] = jnp.dot(h1, w2_ref[...], preferred_element_type=jnp.float32) * dis


def _tc3_body(p_ref, g2_ref, dis_ref, b2_ref, out_ref):
    z = (p_ref[0] + p_ref[1] + g2_ref[...]) * dis_ref[...] + b2_ref[...]
    m = jnp.max(z, axis=1, keepdims=True)
    zs = z - m
    out_ref[...] = zs - jnp.log(jnp.sum(jnp.exp(zs), axis=1, keepdims=True))


def _tc1(x, w1, deg_parts):
    grid = N // BM
    return pl.pallas_call(
        _tc1_body,
        grid=(grid,),
        in_specs=[
            pl.BlockSpec((BM, D_IN), lambda i: (i, 0)),
            pl.BlockSpec((D_IN, D_HID), lambda i: (0, 0)),
            pl.BlockSpec((NC, BM, 16), lambda i: (0, i, 0)),
        ],
        out_specs=[
            pl.BlockSpec((BM, D_HID), lambda i: (i, 0)),
            pl.BlockSpec((BM, 1), lambda i: (i, 0)),
        ],
        out_shape=[
            jax.ShapeDtypeStruct((N, D_HID), jnp.float32),
            jax.ShapeDtypeStruct((N, 1), jnp.float32),
        ],
    )(x, w1, deg_parts)


def _tc2(parts1, g1, dis, b1, w2):
    grid = N // BM
    return pl.pallas_call(
        _tc2_body,
        grid=(grid,),
        in_specs=[
            pl.BlockSpec((NC, BM, D_HID), lambda i: (0, i, 0)),
            pl.BlockSpec((BM, D_HID), lambda i: (i, 0)),
            pl.BlockSpec((BM, 1), lambda i: (i, 0)),
            pl.BlockSpec((1, D_HID), lambda i: (0, 0)),
            pl.BlockSpec((D_HID, D_CLS), lambda i: (0, 0)),
        ],
        out_specs=pl.BlockSpec((BM, D_CLS), lambda i: (i, 0)),
        out_shape=jax.ShapeDtypeStruct((N, D_CLS), jnp.float32),
    )(parts1, g1, dis, b1, w2)


def _tc3(parts2, g2, dis, b2):
    grid = N // BM
    return pl.pallas_call(
        _tc3_body,
        grid=(grid,),
        in_specs=[
            pl.BlockSpec((NC, BM, D_CLS), lambda i: (0, i, 0)),
            pl.BlockSpec((BM, D_CLS), lambda i: (i, 0)),
            pl.BlockSpec((BM, 1), lambda i: (i, 0)),
            pl.BlockSpec((1, D_CLS), lambda i: (0, 0)),
        ],
        out_specs=pl.BlockSpec((BM, D_CLS), lambda i: (i, 0)),
        out_shape=jax.ShapeDtypeStruct((N, D_CLS), jnp.float32),
    )(parts2, g2, dis, b2)


@jax.jit
def kernel(x, edge_index, W1, b1, W2, b2):
    pad = E_PAD - E
    pad_ar = jnp.arange(pad, dtype=jnp.int32)
    src_p = jnp.concatenate([edge_index[0], pad_ar % N]).reshape(NW, CPT, CHUNK)
    dst_p = jnp.concatenate(
        [edge_index[1], N + pad_ar % (ACC_ROWS - N)]).reshape(NW, CPT, CHUNK)

    ones16 = jnp.ones((CHUNK, 16), jnp.float32)
    zeros16 = jnp.zeros((ACC_ROWS, 16), jnp.float32)
    zeros64 = jnp.zeros((ACC_ROWS, D_HID), jnp.float32)
    zeros40 = jnp.zeros((ACC_ROWS, D_CLS), jnp.float32)

    deg_parts = _deg_kernel(dst_p, ones16, zeros16)
    g1, dis = _tc1(x, W1, deg_parts)
    parts1 = _scatter64(src_p, dst_p, g1, zeros64)
    g2 = _tc2(parts1, g1, dis, b1.reshape(1, D_HID), W2)
    parts2 = _scatter40(src_p, dst_p, g2, zeros40)
    return _tc3(parts2, g2, dis, b2.reshape(1, D_CLS))
